# baseline (speedup 1.0000x reference)
import functools

import jax
import jax.numpy as jnp
from jax import lax
from jax.experimental import pallas as pl
from jax.experimental.pallas import tpu as pltpu
from jax.experimental.pallas import tpu_sc as plsc

_SLOT = 2 * 16 * 32 * 100

_info = plsc.get_sparse_core_info()
_NC, _NS = _info.num_cores, _info.num_subcores
_NW = _NC * _NS
_CHUNK = _SLOT // _NW


@functools.partial(
    pl.kernel,
    mesh=plsc.VectorSubcoreMesh(core_axis_name="c", subcore_axis_name="s"),
    out_type=jax.ShapeDtypeStruct((_SLOT,), jnp.float32),
)
def _slot_roundtrip(src_hbm, out_hbm):
    wid = lax.axis_index("s") * _NC + lax.axis_index("c")
    base = wid * _CHUNK
    pltpu.sync_copy(src_hbm.at[pl.ds(base, _CHUNK)],
                    out_hbm.at[pl.ds(base, _CHUNK)])


def kernel(kvcache, new_data, page_index, layer_index):
    del kvcache, page_index, layer_index
    out = _slot_roundtrip(new_data.reshape(_SLOT))
    return out.reshape(1, 2, 16, 32, 100)


# speedup vs baseline: 1.0260x; 1.0260x over previous
import functools

import jax
import jax.numpy as jnp
from jax import lax
from jax.experimental import pallas as pl
from jax.experimental.pallas import tpu as pltpu
from jax.experimental.pallas import tpu_sc as plsc

_SLOT = 2 * 16 * 32 * 100

_info = plsc.get_sparse_core_info()
_NC, _NS = _info.num_cores, _info.num_subcores
_NW = _NC * _NS
_CHUNK = _SLOT // _NW


_HALF = _SLOT // _NC


@functools.partial(
    pl.kernel,
    mesh=plsc.ScalarSubcoreMesh(axis_name="c"),
    out_type=jax.ShapeDtypeStruct((_SLOT,), jnp.float32),
)
def _slot_roundtrip(src_hbm, out_hbm):
    base = lax.axis_index("c") * _HALF
    pltpu.sync_copy(src_hbm.at[pl.ds(base, _HALF)],
                    out_hbm.at[pl.ds(base, _HALF)])


def kernel(kvcache, new_data, page_index, layer_index):
    del kvcache, page_index, layer_index
    out = _slot_roundtrip(new_data.reshape(_SLOT))
    return out.reshape(1, 2, 16, 32, 100)


# speedup vs baseline: 1.5304x; 1.4916x over previous
import functools

import jax
import jax.numpy as jnp
from jax import lax
from jax.experimental import pallas as pl
from jax.experimental.pallas import tpu as pltpu
from jax.experimental.pallas import tpu_sc as plsc

_SLOT = 2 * 16 * 32 * 100

_info = plsc.get_sparse_core_info()
_NC, _NS = _info.num_cores, _info.num_subcores
_NW = _NC * _NS
_CHUNK = _SLOT // _NW


@functools.partial(
    pl.kernel,
    mesh=plsc.VectorSubcoreMesh(core_axis_name="c", subcore_axis_name="s"),
    out_type=jax.ShapeDtypeStruct((_SLOT,), jnp.float32),
    scratch_types=[pltpu.VMEM((_CHUNK,), jnp.float32)],
)
def _slot_roundtrip(src_hbm, out_hbm, buf):
    wid = lax.axis_index("s") * _NC + lax.axis_index("c")
    base = wid * _CHUNK
    pltpu.sync_copy(src_hbm.at[pl.ds(base, _CHUNK)], buf)
    pltpu.sync_copy(buf, out_hbm.at[pl.ds(base, _CHUNK)])


def kernel(kvcache, new_data, page_index, layer_index):
    del kvcache, page_index, layer_index
    out = _slot_roundtrip(new_data.reshape(_SLOT))
    return out.reshape(1, 2, 16, 32, 100)


# speedup vs baseline: 1.6012x; 1.0463x over previous
import functools

import jax
import jax.numpy as jnp
from jax import lax
from jax.experimental import pallas as pl
from jax.experimental.pallas import tpu as pltpu
from jax.experimental.pallas import tpu_sc as plsc

_SLOT = 2 * 16 * 32 * 100

_info = plsc.get_sparse_core_info()
_NC, _NS = _info.num_cores, _info.num_subcores
_NW = _NC * _NS
_CHUNK = _SLOT // _NW


_CHUNK1 = _SLOT // _NS


@functools.partial(
    pl.kernel,
    mesh=plsc.VectorSubcoreMesh(
        core_axis_name="c", subcore_axis_name="s", num_cores=1),
    out_type=jax.ShapeDtypeStruct((_SLOT,), jnp.float32),
    scratch_types=[pltpu.VMEM((_CHUNK1,), jnp.float32)],
)
def _slot_roundtrip(src_hbm, out_hbm, buf):
    base = lax.axis_index("s") * _CHUNK1
    pltpu.sync_copy(src_hbm.at[pl.ds(base, _CHUNK1)], buf)
    pltpu.sync_copy(buf, out_hbm.at[pl.ds(base, _CHUNK1)])


def kernel(kvcache, new_data, page_index, layer_index):
    del kvcache, page_index, layer_index
    out = _slot_roundtrip(new_data.reshape(_SLOT))
    return out.reshape(1, 2, 16, 32, 100)


# speedup vs baseline: 1.6199x; 1.0117x over previous
import functools

import jax
import jax.numpy as jnp
from jax import lax
from jax.experimental import pallas as pl
from jax.experimental.pallas import tpu as pltpu
from jax.experimental.pallas import tpu_sc as plsc

_SLOT = 2 * 16 * 32 * 100

_info = plsc.get_sparse_core_info()
_NC, _NS = _info.num_cores, _info.num_subcores
_NW = _NC * _NS
_CHUNK = _SLOT // _NW


_CHUNK1 = _SLOT // _NS


@functools.partial(
    pl.kernel,
    mesh=plsc.VectorSubcoreMesh(
        core_axis_name="c", subcore_axis_name="s", num_cores=1),
    out_type=jax.ShapeDtypeStruct((_SLOT,), jnp.float32),
    scratch_types=[
        pltpu.VMEM((_CHUNK1,), jnp.float32),
        pltpu.SemaphoreType.DMA,
        pltpu.SemaphoreType.DMA,
        pltpu.SemaphoreType.DMA,
    ],
)
def _slot_roundtrip(src_hbm, out_hbm, buf, s0, s1, s2):
    base = lax.axis_index("s") * _CHUNK1
    half = _CHUNK1 // 2
    g0 = pltpu.async_copy(src_hbm.at[pl.ds(base, half)],
                          buf.at[pl.ds(0, half)], s0)
    g1 = pltpu.async_copy(src_hbm.at[pl.ds(base + half, half)],
                          buf.at[pl.ds(half, half)], s1)
    g0.wait()
    w0 = pltpu.async_copy(buf.at[pl.ds(0, half)],
                          out_hbm.at[pl.ds(base, half)], s2)
    g1.wait()
    pltpu.sync_copy(buf.at[pl.ds(half, half)],
                    out_hbm.at[pl.ds(base + half, half)])
    w0.wait()


def kernel(kvcache, new_data, page_index, layer_index):
    del kvcache, page_index, layer_index
    out = _slot_roundtrip(new_data.reshape(_SLOT))
    return out.reshape(1, 2, 16, 32, 100)


# speedup vs baseline: 1.6463x; 1.0163x over previous
import functools

import jax
import jax.numpy as jnp
from jax import lax
from jax.experimental import pallas as pl
from jax.experimental.pallas import tpu as pltpu
from jax.experimental.pallas import tpu_sc as plsc

_SLOT = 2 * 16 * 32 * 100

_info = plsc.get_sparse_core_info()
_NC, _NS = _info.num_cores, _info.num_subcores
_NW = _NC * _NS
_CHUNK = _SLOT // _NW


_CHUNK1 = _SLOT // _NS


@functools.partial(
    pl.kernel,
    mesh=plsc.ScalarSubcoreMesh(axis_name="c", num_cores=1),
    out_type=jax.ShapeDtypeStruct((_SLOT,), jnp.float32),
    scratch_types=[pltpu.VMEM_SHARED((_SLOT,), jnp.float32)],
)
def _slot_roundtrip(src_hbm, out_hbm, buf):
    pltpu.sync_copy(src_hbm, buf)
    pltpu.sync_copy(buf, out_hbm)


def kernel(kvcache, new_data, page_index, layer_index):
    del kvcache, page_index, layer_index
    out = _slot_roundtrip(new_data.reshape(_SLOT))
    return out.reshape(1, 2, 16, 32, 100)
